# baseline (speedup 1.0000x reference)
import functools

import jax
import jax.numpy as jnp
from jax import lax
from jax.experimental import pallas as pl
from jax.experimental.pallas import tpu as pltpu
from jax.experimental.pallas import tpu_sc as plsc

NC = 2
NS = 16
LANES = 16
NW = NC * NS
CH = 16

_GATHER_1D = lax.GatherDimensionNumbers(
    offset_dims=(), collapsed_slice_dims=(0,), start_index_map=(0,))


def _lane_perm(x, perm):
  return lax.gather(x, perm[:, None], _GATHER_1D, slice_sizes=(1,),
                    mode=lax.GatherScatterMode.PROMISE_IN_BOUNDS)


def _sc_embed_ln(ids, tts, word_table, pos_table, tok_table, gamma, beta,
                 *, seq_len):
  n_tok = ids.shape[0]
  dim = word_table.shape[1]
  per_w = n_tok // NW
  n_chunks = per_w // CH
  nvec = dim // LANES
  inv_dim = 1.0 / dim

  mesh = plsc.VectorSubcoreMesh(
      core_axis_name="c", subcore_axis_name="s",
      num_cores=NC, num_subcores=NS)

  @functools.partial(
      pl.kernel,
      out_type=jax.ShapeDtypeStruct((n_tok, dim), jnp.float32),
      mesh=mesh,
      scratch_types=[
          pltpu.VMEM((per_w,), jnp.int32),
          pltpu.VMEM((per_w,), jnp.int32),
          pltpu.VMEM((CH, dim), jnp.float32),
          pltpu.VMEM((CH, dim), jnp.float32),
          pltpu.VMEM((CH, dim), jnp.float32),
          pltpu.VMEM((CH, dim), jnp.float32),
          pltpu.VMEM((CH, dim), jnp.float32),
          pltpu.VMEM((CH, dim), jnp.float32),
          pltpu.VMEM((dim,), jnp.float32),
          pltpu.VMEM((dim,), jnp.float32),
          pltpu.VMEM((dim,), jnp.float32),
          pltpu.VMEM((dim,), jnp.float32),
          pltpu.VMEM((2, CH * LANES), jnp.float32),
          pltpu.SemaphoreType.DMA,
          pltpu.SemaphoreType.DMA,
          pltpu.SemaphoreType.DMA,
          pltpu.SemaphoreType.DMA,
      ],
  )
  def body(ids_hbm, tts_hbm, word_hbm, pos_hbm, tok_hbm, gamma_hbm, beta_hbm,
           out_hbm, idxa, tta, r0, r1, p0, p1, o0, o1,
           tokb_v, tokd_v, gamma_v, beta_v, stats_v, sg0, sg1, so0, so1):
    wid = lax.axis_index("s") * NC + lax.axis_index("c")
    base = wid * per_w

    def prologue_descs():
      return [
          pltpu.make_async_copy(gamma_hbm, gamma_v, sg0),
          pltpu.make_async_copy(beta_hbm, beta_v, sg0),
          pltpu.make_async_copy(tok_hbm.at[0], tokb_v, sg0),
          pltpu.make_async_copy(tok_hbm.at[1], tokd_v, sg0),
          pltpu.make_async_copy(ids_hbm.at[pl.ds(base, per_w)], idxa, sg0),
          pltpu.make_async_copy(tts_hbm.at[pl.ds(base, per_w)], tta, sg0),
      ]

    for d in prologue_descs():
      d.start()
    for d in prologue_descs():
      d.wait()
    for j in range(nvec):
      sl = pl.ds(j * LANES, LANES)
      tokd_v[sl] = tokd_v[sl] - tokb_v[sl]

    slots = ((r0, p0, o0, sg0, so0), (r1, p1, o1, sg1, so1))

    def g_descs(c, rows, pos, sg):
      tb = base + c * CH
      s_b = lax.rem(tb, seq_len)
      d_pos = pltpu.make_async_copy(pos_hbm.at[pl.ds(s_b, CH)], pos, sg)
      d_wrd = pltpu.make_async_copy(word_hbm.at[idxa.at[pl.ds(c * CH, CH)]],
                                    rows, sg)
      return d_pos, d_wrd

    def issue_g(c, rows, pos, sg):
      for d in g_descs(c, rows, pos, sg):
        d.start()

    def wait_g(c, rows, pos, sg):
      for d in g_descs(c, rows, pos, sg):
        d.wait()

    def out_desc(c, outb, so):
      tb = base + c * CH
      return pltpu.make_async_copy(outb, out_hbm.at[pl.ds(tb, CH)], so)

    GRP = 4
    n_grp = nvec // GRP

    QT = 4

    def compute(c, rows, pos, outb):
      ttv16 = tta[pl.ds(c * CH, CH)]

      def q1_body(q, _):
        t0 = q * QT
        ttfs = [
            _lane_perm(ttv16, jnp.full((LANES,), t0 + i, jnp.int32)).astype(
                jnp.float32) for i in range(QT)
        ]
        accs = [jnp.zeros((LANES,), jnp.float32) for _ in range(QT)]
        accq = [jnp.zeros((LANES,), jnp.float32) for _ in range(QT)]

        def load1(j):
          sl = pl.ds(j * LANES, LANES)
          return (tokb_v[sl], tokd_v[sl],
                  [rows[t0 + i, sl] for i in range(QT)],
                  [pos[t0 + i, sl] for i in range(QT)], sl)

        def consume1(vals):
          tb, td, ws, ps, sl = vals
          for i in range(QT):
            x = (ws[i] + ps[i]) + (tb + ttfs[i] * td)
            outb[t0 + i, sl] = x
            accs[i] = accs[i] + x
            accq[i] = accq[i] + x * x

        prev = load1(0)
        for j in range(1, nvec):
          cur = load1(j)
          consume1(prev)
          prev = cur
        consume1(prev)

        lanes = lax.iota(jnp.int32, LANES)
        for sh in (8, 4, 2, 1):
          perm = lanes ^ sh
          for i in range(QT):
            accs[i] = accs[i] + _lane_perm(accs[i], perm)
          for i in range(QT):
            accq[i] = accq[i] + _lane_perm(accq[i], perm)
        mus = [accs[i] * inv_dim for i in range(QT)]
        vvs = [accq[i] * inv_dim - mus[i] * mus[i] + 1e-12 for i in range(QT)]
        ys = [
            lax.bitcast_convert_type(
                jnp.int32(0x5F3759DF) -
                (lax.bitcast_convert_type(vvs[i], jnp.int32) >> 1),
                jnp.float32) for i in range(QT)
        ]
        for _ in range(2):
          ys = [ys[i] * (1.5 - 0.5 * vvs[i] * ys[i] * ys[i])
                for i in range(QT)]
        for i in range(QT):
          st = pl.ds((t0 + i) * LANES, LANES)
          stats_v[0, st] = mus[i]
          stats_v[1, st] = ys[i]
        return 0

      lax.fori_loop(0, CH // QT, q1_body, 0)

      def quarter_body(q, _):
        t0 = q * QT
        mus = [stats_v[0, pl.ds((t0 + i) * LANES, LANES)] for i in range(QT)]
        ys = [stats_v[1, pl.ds((t0 + i) * LANES, LANES)] for i in range(QT)]

        def load2(j):
          sl = pl.ds(j * LANES, LANES)
          return (gamma_v[sl], beta_v[sl],
                  [outb[t0 + i, sl] for i in range(QT)], sl)

        def consume2(vals):
          gmm, bta, xs, sl = vals
          for i in range(QT):
            outb[t0 + i, sl] = ((xs[i] - mus[i]) * ys[i]) * gmm + bta

        prev = load2(0)
        for j in range(1, nvec):
          cur = load2(j)
          consume2(prev)
          prev = cur
        consume2(prev)
        return 0

      lax.fori_loop(0, CH // QT, quarter_body, 0)

    issue_g(0, r0, p0, sg0)
    issue_g(1, r1, p1, sg1)

    def pair_body(k, _):
      for b in (0, 1):
        rows, pos, outb, sg, so = slots[b]
        c = 2 * k + b
        wait_g(c, rows, pos, sg)

        @pl.when(c >= 2)
        def _():
          out_desc(c, outb, so).wait()

        compute(c, rows, pos, outb)
        out_desc(c, outb, so).start()

        @pl.when(c + 2 < n_chunks)
        def _():
          issue_g(c + 2, rows, pos, sg)
      return 0

    lax.fori_loop(0, n_chunks // 2, pair_body, 0)
    out_desc(n_chunks - 2, o0, so0).wait()
    out_desc(n_chunks - 1, o1, so1).wait()

  return body(ids, tts, word_table, pos_table, tok_table, gamma, beta)


def kernel(input_ids, token_type_ids, word_table, pos_table, tok_table,
           gamma, beta):
  b, s = input_ids.shape
  dim = word_table.shape[1]
  ids = input_ids.reshape(b * s).astype(jnp.int32)
  tts = token_type_ids.reshape(b * s).astype(jnp.int32)
  out = _sc_embed_ln(ids, tts, word_table.astype(jnp.float32),
                     pos_table.astype(jnp.float32),
                     tok_table.astype(jnp.float32),
                     gamma.astype(jnp.float32), beta.astype(jnp.float32),
                     seq_len=s)
  return out.reshape(b, s, dim)


# speedup vs baseline: 1.0458x; 1.0458x over previous
import functools

import jax
import jax.numpy as jnp
from jax import lax
from jax.experimental import pallas as pl
from jax.experimental.pallas import tpu as pltpu
from jax.experimental.pallas import tpu_sc as plsc

NC = 2
NS = 16
LANES = 16
NW = NC * NS
CH = 16

_GATHER_1D = lax.GatherDimensionNumbers(
    offset_dims=(), collapsed_slice_dims=(0,), start_index_map=(0,))


def _lane_perm(x, perm):
  return lax.gather(x, perm[:, None], _GATHER_1D, slice_sizes=(1,),
                    mode=lax.GatherScatterMode.PROMISE_IN_BOUNDS)


def _sc_embed_ln(ids, tts, word_table, pos_table, tok_table, gamma, beta,
                 *, seq_len):
  n_tok = ids.shape[0]
  dim = word_table.shape[1]
  per_w = n_tok // NW
  n_chunks = per_w // CH
  nvec = dim // LANES
  inv_dim = 1.0 / dim

  mesh = plsc.VectorSubcoreMesh(
      core_axis_name="c", subcore_axis_name="s",
      num_cores=NC, num_subcores=NS)

  @functools.partial(
      pl.kernel,
      out_type=jax.ShapeDtypeStruct((n_tok, dim), jnp.float32),
      mesh=mesh,
      scratch_types=[
          pltpu.VMEM((per_w,), jnp.int32),
          pltpu.VMEM((per_w,), jnp.int32),
          pltpu.VMEM((CH, dim), jnp.float32),
          pltpu.VMEM((CH, dim), jnp.float32),
          pltpu.VMEM((CH, dim), jnp.float32),
          pltpu.VMEM((CH, dim), jnp.float32),
          pltpu.VMEM((CH, dim), jnp.float32),
          pltpu.VMEM((CH, dim), jnp.float32),
          pltpu.VMEM((dim,), jnp.float32),
          pltpu.VMEM((dim,), jnp.float32),
          pltpu.VMEM((dim,), jnp.float32),
          pltpu.VMEM((dim,), jnp.float32),
          pltpu.VMEM((2, CH * LANES), jnp.float32),
          pltpu.SemaphoreType.DMA,
          pltpu.SemaphoreType.DMA,
          pltpu.SemaphoreType.DMA,
          pltpu.SemaphoreType.DMA,
      ],
  )
  def body(ids_hbm, tts_hbm, word_hbm, pos_hbm, tok_hbm, gamma_hbm, beta_hbm,
           out_hbm, idxa, tta, r0, r1, p0, p1, o0, o1,
           tokb_v, tokd_v, gamma_v, beta_v, stats_v, sg0, sg1, so0, so1):
    wid = lax.axis_index("s") * NC + lax.axis_index("c")
    base = wid * per_w

    def prologue_descs():
      return [
          pltpu.make_async_copy(gamma_hbm, gamma_v, sg0),
          pltpu.make_async_copy(beta_hbm, beta_v, sg0),
          pltpu.make_async_copy(tok_hbm.at[0], tokb_v, sg0),
          pltpu.make_async_copy(tok_hbm.at[1], tokd_v, sg0),
          pltpu.make_async_copy(ids_hbm.at[pl.ds(base, per_w)], idxa, sg0),
          pltpu.make_async_copy(tts_hbm.at[pl.ds(base, per_w)], tta, sg0),
      ]

    for d in prologue_descs():
      d.start()
    for d in prologue_descs():
      d.wait()
    for j in range(nvec):
      sl = pl.ds(j * LANES, LANES)
      tokd_v[sl] = tokd_v[sl] - tokb_v[sl]

    slots = ((r0, p0, o0, sg0, so0), (r1, p1, o1, sg1, so1))

    def g_descs(c, rows, pos, sg):
      tb = base + c * CH
      s_b = lax.rem(tb, seq_len)
      d_pos = pltpu.make_async_copy(pos_hbm.at[pl.ds(s_b, CH)], pos, sg)
      d_wrd = pltpu.make_async_copy(word_hbm.at[idxa.at[pl.ds(c * CH, CH)]],
                                    rows, sg)
      return d_pos, d_wrd

    def issue_g(c, rows, pos, sg):
      for d in g_descs(c, rows, pos, sg):
        d.start()

    def wait_g(c, rows, pos, sg):
      for d in g_descs(c, rows, pos, sg):
        d.wait()

    def out_desc(c, outb, so):
      tb = base + c * CH
      return pltpu.make_async_copy(outb, out_hbm.at[pl.ds(tb, CH)], so)

    GRP = 4
    n_grp = nvec // GRP

    QT = 4

    def compute(c, rows, pos, outb):
      ttv16 = tta[pl.ds(c * CH, CH)]

      QP = 2

      def q1_body(q, _):
        t0 = q * QP
        ttfs = [
            _lane_perm(ttv16, jnp.full((LANES,), t0 + i, jnp.int32)).astype(
                jnp.float32) for i in range(QP)
        ]
        accs = [jnp.zeros((LANES,), jnp.float32) for _ in range(QP)]
        accq = [jnp.zeros((LANES,), jnp.float32) for _ in range(QP)]

        def load1(j):
          sl = pl.ds(j * LANES, LANES)
          return (tokb_v[sl], tokd_v[sl],
                  [rows[t0 + i, sl] for i in range(QP)],
                  [pos[t0 + i, sl] for i in range(QP)], sl)

        def consume1(vals):
          tb, td, ws, ps, sl = vals
          for i in range(QP):
            x = (ws[i] + ps[i]) + (tb + ttfs[i] * td)
            outb[t0 + i, sl] = x
            accs[i] = accs[i] + x
            accq[i] = accq[i] + x * x

        prev = load1(0)
        for j in range(1, nvec):
          cur = load1(j)
          consume1(prev)
          prev = cur
        consume1(prev)

        lanes = lax.iota(jnp.int32, LANES)
        for sh in (8, 4, 2, 1):
          perm = lanes ^ sh
          for i in range(QP):
            accs[i] = accs[i] + _lane_perm(accs[i], perm)
          for i in range(QP):
            accq[i] = accq[i] + _lane_perm(accq[i], perm)
        mus = [accs[i] * inv_dim for i in range(QP)]
        vvs = [accq[i] * inv_dim - mus[i] * mus[i] + 1e-12 for i in range(QP)]
        ys = [
            lax.bitcast_convert_type(
                jnp.int32(0x5F3759DF) -
                (lax.bitcast_convert_type(vvs[i], jnp.int32) >> 1),
                jnp.float32) for i in range(QP)
        ]
        for _ in range(2):
          ys = [ys[i] * (1.5 - 0.5 * vvs[i] * ys[i] * ys[i])
                for i in range(QP)]
        for i in range(QP):
          st = pl.ds((t0 + i) * LANES, LANES)
          stats_v[0, st] = mus[i]
          stats_v[1, st] = ys[i]
        return 0

      lax.fori_loop(0, CH // QP, q1_body, 0)

      def quarter_body(q, _):
        t0 = q * QT
        mus = [stats_v[0, pl.ds((t0 + i) * LANES, LANES)] for i in range(QT)]
        ys = [stats_v[1, pl.ds((t0 + i) * LANES, LANES)] for i in range(QT)]

        def load2(j):
          sl = pl.ds(j * LANES, LANES)
          return (gamma_v[sl], beta_v[sl],
                  [outb[t0 + i, sl] for i in range(QT)], sl)

        def consume2(vals):
          gmm, bta, xs, sl = vals
          for i in range(QT):
            outb[t0 + i, sl] = ((xs[i] - mus[i]) * ys[i]) * gmm + bta

        prev = load2(0)
        for j in range(1, nvec):
          cur = load2(j)
          consume2(prev)
          prev = cur
        consume2(prev)
        return 0

      lax.fori_loop(0, CH // QT, quarter_body, 0)

    issue_g(0, r0, p0, sg0)
    issue_g(1, r1, p1, sg1)

    def pair_body(k, _):
      for b in (0, 1):
        rows, pos, outb, sg, so = slots[b]
        c = 2 * k + b
        wait_g(c, rows, pos, sg)

        @pl.when(c >= 2)
        def _():
          out_desc(c, outb, so).wait()

        compute(c, rows, pos, outb)
        out_desc(c, outb, so).start()

        @pl.when(c + 2 < n_chunks)
        def _():
          issue_g(c + 2, rows, pos, sg)
      return 0

    lax.fori_loop(0, n_chunks // 2, pair_body, 0)
    out_desc(n_chunks - 2, o0, so0).wait()
    out_desc(n_chunks - 1, o1, so1).wait()

  return body(ids, tts, word_table, pos_table, tok_table, gamma, beta)


def kernel(input_ids, token_type_ids, word_table, pos_table, tok_table,
           gamma, beta):
  b, s = input_ids.shape
  dim = word_table.shape[1]
  ids = input_ids.reshape(b * s).astype(jnp.int32)
  tts = token_type_ids.reshape(b * s).astype(jnp.int32)
  out = _sc_embed_ln(ids, tts, word_table.astype(jnp.float32),
                     pos_table.astype(jnp.float32),
                     tok_table.astype(jnp.float32),
                     gamma.astype(jnp.float32), beta.astype(jnp.float32),
                     seq_len=s)
  return out.reshape(b, s, dim)
